# baseline (speedup 1.0000x reference)
import functools

import jax
import jax.numpy as jnp
from jax import lax
from jax.experimental import pallas as pl
from jax.experimental.pallas import tpu as pltpu
from jax.experimental.pallas import tpu_sc as plsc

NN = 10000
NP = 10240
EE = 160000
CHUNK = 128
NSUB = 16
CPT = 80
NCH = NSUB * CPT
NCHP = NCH + 8
DEG_CPT = NCH // 2 // NSUB
SLAB = NP // NSUB
RB = 1024
NRB = NP // RB

_sc_mesh = plsc.VectorSubcoreMesh(core_axis_name="c", subcore_axis_name="s")


@functools.partial(
    pl.kernel,
    out_type=jax.ShapeDtypeStruct((2 * NP, 16), jnp.float32),
    mesh=_sc_mesh,
    scratch_types=[
        pltpu.VMEM_SHARED((NP, 16), jnp.float32),
        pltpu.VMEM((SLAB, 16), jnp.float32),
        pltpu.VMEM((CHUNK, 16), jnp.float32),
        pltpu.VMEM((DEG_CPT, CHUNK), jnp.int32),
    ],
)
def _sc_degree(dst2_hbm, out_hbm, acc_sh, zero_v, ones_v, idx_v):
    c = lax.axis_index("c")
    s = lax.axis_index("s")
    slab = s * SLAB
    zero16 = jnp.zeros((16,), jnp.float32)
    ones16 = jnp.ones((16,), jnp.float32)

    def fill_zero(j, carry):
        zero_v[j, pl.ds(0, 16)] = zero16
        return carry

    lax.fori_loop(0, SLAB, fill_zero, 0)

    def fill_one(j, carry):
        ones_v[j, pl.ds(0, 16)] = ones16
        return carry

    lax.fori_loop(0, CHUNK, fill_one, 0)

    pltpu.sync_copy(dst2_hbm.at[pl.ds(c * (NCH // 2) + s * DEG_CPT, DEG_CPT)],
                    idx_v)
    pltpu.sync_copy(zero_v, acc_sh.at[pl.ds(slab, SLAB)])
    plsc.subcore_barrier()

    def count(i, carry):
        pltpu.sync_copy(ones_v, acc_sh.at[idx_v.at[i]], add=True)
        return carry

    lax.fori_loop(0, DEG_CPT, count, 0)
    plsc.subcore_barrier()

    pltpu.sync_copy(acc_sh.at[pl.ds(slab, SLAB)],
                    out_hbm.at[pl.ds(c * NP + slab, SLAB)])


@functools.partial(
    pl.kernel,
    out_type=jax.ShapeDtypeStruct((2 * NP, 128), jnp.float32),
    mesh=_sc_mesh,
    scratch_types=[
        pltpu.VMEM_SHARED((NP, 128), jnp.float32),
        pltpu.VMEM((CPT, CHUNK), jnp.int32),
        pltpu.VMEM((CPT, CHUNK), jnp.int32),
        pltpu.VMEM((CHUNK, 128), jnp.float32),
        pltpu.SemaphoreType.DMA,
    ],
)
def _sc_propagate(src3_hbm, dst2_hbm, y_hbm, out_hbm,
                  acc_sh, src_v, dst_v, rows_v, sem0):
    c = lax.axis_index("c")
    s = lax.axis_index("s")
    base = c * NP
    slab = s * SLAB
    cstart = s * CPT

    pltpu.sync_copy(src3_hbm.at[c].at[pl.ds(cstart, CPT)], src_v)
    pltpu.sync_copy(dst2_hbm.at[pl.ds(cstart, CPT)], dst_v)
    pltpu.sync_copy(y_hbm.at[pl.ds(base + slab, SLAB)],
                    acc_sh.at[pl.ds(slab, SLAB)])
    plsc.subcore_barrier()

    def body(ch, carry):
        pltpu.async_copy(y_hbm.at[src_v.at[ch]], rows_v, sem0).wait()
        pltpu.sync_copy(rows_v, acc_sh.at[dst_v.at[ch]], add=True)
        return carry

    lax.fori_loop(0, CPT, body, 0)
    plsc.subcore_barrier()
    pltpu.sync_copy(acc_sh.at[pl.ds(slab, SLAB)],
                    out_hbm.at[pl.ds(base + slab, SLAB)])


def _dinv(dlo_ref, dhi_ref):
    return lax.rsqrt(dlo_ref[:, 0:1] + dhi_ref[:, 0:1] + 1.0)


def _mm1_body(x_ref, w_ref, dlo_ref, dhi_ref, o_ref):
    y = jnp.dot(x_ref[...], w_ref[...], preferred_element_type=jnp.float32,
                precision=lax.Precision.HIGHEST)
    o_ref[...] = y * _dinv(dlo_ref, dhi_ref)


def _mm2_body(zlo_ref, zhi_ref, w_ref, b_ref, dlo_ref, dhi_ref, o_ref):
    dinv = _dinv(dlo_ref, dhi_ref)
    z = jnp.concatenate([zlo_ref[...], zhi_ref[...]], axis=1)
    h = jnp.maximum(z * dinv + b_ref[...], 0.0)
    o_ref[...] = jnp.dot(h, w_ref[...], preferred_element_type=jnp.float32,
                         precision=lax.Precision.HIGHEST) * dinv


def _fin_body(zlo_ref, zhi_ref, bmu_ref, bls_ref, dlo_ref, dhi_ref,
              mu_ref, ls_ref):
    dinv = _dinv(dlo_ref, dhi_ref)
    mu_ref[...] = zlo_ref[...] * dinv + bmu_ref[...]
    ls_ref[...] = zhi_ref[...] * dinv + bls_ref[...]


def _deg_specs():
    return [
        pl.BlockSpec((RB, 16), lambda i, j: (i, 0)),
        pl.BlockSpec((RB, 16), lambda i, j: (NRB + i, 0)),
    ]


def _tc_mm1(x, w1, degs):
    return pl.pallas_call(
        _mm1_body,
        grid=(NRB, 2),
        in_specs=[
            pl.BlockSpec((RB, 256), lambda i, j: (i, 0)),
            pl.BlockSpec((256, 128), lambda i, j: (0, j)),
            *_deg_specs(),
        ],
        out_specs=pl.BlockSpec((RB, 128), lambda i, j: (j * NRB + i, 0)),
        out_shape=jax.ShapeDtypeStruct((2 * NP, 128), jnp.float32),
    )(x, w1, degs, degs)


def _tc_mm2(z1, wcat, b1r, degs):
    return pl.pallas_call(
        _mm2_body,
        grid=(NRB, 2),
        in_specs=[
            pl.BlockSpec((RB, 128), lambda i, j: (i, 0)),
            pl.BlockSpec((RB, 128), lambda i, j: (NRB + i, 0)),
            pl.BlockSpec((256, 128), lambda i, j: (0, j)),
            pl.BlockSpec((1, 256), lambda i, j: (0, 0)),
            *_deg_specs(),
        ],
        out_specs=pl.BlockSpec((RB, 128), lambda i, j: (j * NRB + i, 0)),
        out_shape=jax.ShapeDtypeStruct((2 * NP, 128), jnp.float32),
    )(z1, z1, wcat, b1r, degs, degs)


def _tc_final(z2, bmur, blsr, degs):
    spec = pl.BlockSpec((RB, 128), lambda i: (i, 0))
    return pl.pallas_call(
        _fin_body,
        grid=(NRB,),
        in_specs=[
            pl.BlockSpec((RB, 128), lambda i: (i, 0)),
            pl.BlockSpec((RB, 128), lambda i: (NRB + i, 0)),
            pl.BlockSpec((1, 128), lambda i: (0, 0)),
            pl.BlockSpec((1, 128), lambda i: (0, 0)),
            pl.BlockSpec((RB, 16), lambda i: (i, 0)),
            pl.BlockSpec((RB, 16), lambda i: (NRB + i, 0)),
        ],
        out_specs=[spec, spec],
        out_shape=[
            jax.ShapeDtypeStruct((NP, 128), jnp.float32),
            jax.ShapeDtypeStruct((NP, 128), jnp.float32),
        ],
    )(z2, z2, bmur, blsr, degs, degs)


def kernel(x, edge_index, W1, b1, Wmu, bmu, Wls, bls):
    pad_e = NCHP * CHUNK - EE
    src2 = jnp.concatenate(
        [edge_index[0], jnp.zeros((pad_e,), jnp.int32)]).reshape(NCHP, CHUNK)
    src3 = jnp.stack([src2, src2 + NP])
    dst2 = jnp.concatenate(
        [edge_index[1], jnp.full((pad_e,), NN, jnp.int32)]).reshape(NCHP, CHUNK)
    xp = jnp.pad(x, ((0, NP - NN), (0, 0)))
    wcat = jnp.concatenate([Wmu, Wls], axis=1)
    b1r = b1.reshape(1, 256)
    bmur = bmu.reshape(1, 128)
    blsr = bls.reshape(1, 128)
    degs = _sc_degree(dst2)
    y1 = _tc_mm1(xp, W1, degs)
    z1 = _sc_propagate(src3, dst2, y1)
    y2 = _tc_mm2(z1, wcat, b1r, degs)
    z2 = _sc_propagate(src3, dst2, y2)
    mu, ls = _tc_final(z2, bmur, blsr, degs)
    return mu[:NN], ls[:NN]


# speedup vs baseline: 1.1617x; 1.1617x over previous
import functools

import jax
import jax.numpy as jnp
from jax import lax
from jax.experimental import pallas as pl
from jax.experimental.pallas import tpu as pltpu
from jax.experimental.pallas import tpu_sc as plsc

NN = 10000
NP = 10240
EE = 160000
CHUNK = 128
NSUB = 16
CPT = 80
NCH = NSUB * CPT
NCHP = NCH + 8
DEG_CPT = NCH // 2 // NSUB
SLAB = NP // NSUB
RB = 1024
NRB = NP // RB

_sc_mesh = plsc.VectorSubcoreMesh(core_axis_name="c", subcore_axis_name="s")


@functools.partial(
    pl.kernel,
    out_type=jax.ShapeDtypeStruct((2 * NP, 16), jnp.float32),
    mesh=_sc_mesh,
    scratch_types=[
        pltpu.VMEM_SHARED((NP, 16), jnp.float32),
        pltpu.VMEM((SLAB, 16), jnp.float32),
        pltpu.VMEM((CHUNK, 16), jnp.float32),
        pltpu.VMEM((DEG_CPT, CHUNK), jnp.int32),
    ],
)
def _sc_degree(dst2_hbm, out_hbm, acc_sh, zero_v, ones_v, idx_v):
    c = lax.axis_index("c")
    s = lax.axis_index("s")
    slab = s * SLAB
    zero16 = jnp.zeros((16,), jnp.float32)
    ones16 = jnp.ones((16,), jnp.float32)

    def fill_zero(j, carry):
        zero_v[j, pl.ds(0, 16)] = zero16
        return carry

    lax.fori_loop(0, SLAB, fill_zero, 0)

    def fill_one(j, carry):
        ones_v[j, pl.ds(0, 16)] = ones16
        return carry

    lax.fori_loop(0, CHUNK, fill_one, 0)

    pltpu.sync_copy(dst2_hbm.at[pl.ds(c * (NCH // 2) + s * DEG_CPT, DEG_CPT)],
                    idx_v)
    pltpu.sync_copy(zero_v, acc_sh.at[pl.ds(slab, SLAB)])
    plsc.subcore_barrier()

    def count(i, carry):
        pltpu.sync_copy(ones_v, acc_sh.at[idx_v.at[i]], add=True)
        return carry

    lax.fori_loop(0, DEG_CPT, count, 0)
    plsc.subcore_barrier()

    pltpu.sync_copy(acc_sh.at[pl.ds(slab, SLAB)],
                    out_hbm.at[pl.ds(c * NP + slab, SLAB)])


@functools.partial(
    pl.kernel,
    out_type=jax.ShapeDtypeStruct((2 * NP, 128), jnp.float32),
    mesh=_sc_mesh,
    scratch_types=[
        pltpu.VMEM_SHARED((NP, 128), jnp.float32),
        pltpu.VMEM((CPT // 2, CHUNK), jnp.int32),
        pltpu.VMEM((CPT // 2, CHUNK), jnp.int32),
        pltpu.VMEM((2, CHUNK, 128), jnp.float32),
        pltpu.SemaphoreType.DMA,
        pltpu.SemaphoreType.DMA,
    ],
)
def _sc_propagate(src3_hbm, dst2_hbm, y_hbm, out_hbm,
                  acc_sh, src_v, dst_v, rows_v, sem0, sem1):
    c = lax.axis_index("c")
    s = lax.axis_index("s")
    base = c * NP
    slab = s * SLAB
    cstart = s * CPT
    sems = (sem0, sem1)

    pltpu.sync_copy(y_hbm.at[pl.ds(base + slab, SLAB)],
                    acc_sh.at[pl.ds(slab, SLAB)])
    plsc.subcore_barrier()

    def start_gather(ch, b):
        return pltpu.async_copy(y_hbm.at[src_v.at[ch]], rows_v.at[b], sems[b])

    PCH = CPT // 2
    for ph in range(2):
        pltpu.sync_copy(src3_hbm.at[c].at[pl.ds(cstart + ph * PCH, PCH)],
                        src_v)
        pltpu.sync_copy(dst2_hbm.at[pl.ds(cstart + ph * PCH, PCH)], dst_v)

        handles = {0: start_gather(0, 0)}
        for ch in range(PCH):
            if ch + 1 < PCH:
                handles[ch + 1] = start_gather(ch + 1, (ch + 1) % 2)
            handles.pop(ch).wait()
            pltpu.sync_copy(rows_v.at[ch % 2], acc_sh.at[dst_v.at[ch]],
                            add=True)

    plsc.subcore_barrier()
    pltpu.sync_copy(acc_sh.at[pl.ds(slab, SLAB)],
                    out_hbm.at[pl.ds(base + slab, SLAB)])


def _dinv(dlo_ref, dhi_ref):
    return lax.rsqrt(dlo_ref[:, 0:1] + dhi_ref[:, 0:1] + 1.0)


def _mm1_body(x_ref, w_ref, dlo_ref, dhi_ref, o_ref):
    y = jnp.dot(x_ref[...], w_ref[...], preferred_element_type=jnp.float32,
                precision=lax.Precision.HIGHEST)
    o_ref[...] = y * _dinv(dlo_ref, dhi_ref)


def _mm2_body(zlo_ref, zhi_ref, w_ref, b_ref, dlo_ref, dhi_ref, o_ref):
    dinv = _dinv(dlo_ref, dhi_ref)
    z = jnp.concatenate([zlo_ref[...], zhi_ref[...]], axis=1)
    h = jnp.maximum(z * dinv + b_ref[...], 0.0)
    o_ref[...] = jnp.dot(h, w_ref[...], preferred_element_type=jnp.float32,
                         precision=lax.Precision.HIGHEST) * dinv


def _fin_body(zlo_ref, zhi_ref, bmu_ref, bls_ref, dlo_ref, dhi_ref,
              mu_ref, ls_ref):
    dinv = _dinv(dlo_ref, dhi_ref)
    mu_ref[...] = zlo_ref[...] * dinv + bmu_ref[...]
    ls_ref[...] = zhi_ref[...] * dinv + bls_ref[...]


def _deg_specs():
    return [
        pl.BlockSpec((RB, 16), lambda i, j: (i, 0)),
        pl.BlockSpec((RB, 16), lambda i, j: (NRB + i, 0)),
    ]


def _tc_mm1(x, w1, degs):
    return pl.pallas_call(
        _mm1_body,
        grid=(NRB, 2),
        in_specs=[
            pl.BlockSpec((RB, 256), lambda i, j: (i, 0)),
            pl.BlockSpec((256, 128), lambda i, j: (0, j)),
            *_deg_specs(),
        ],
        out_specs=pl.BlockSpec((RB, 128), lambda i, j: (j * NRB + i, 0)),
        out_shape=jax.ShapeDtypeStruct((2 * NP, 128), jnp.float32),
    )(x, w1, degs, degs)


def _tc_mm2(z1, wcat, b1r, degs):
    return pl.pallas_call(
        _mm2_body,
        grid=(NRB, 2),
        in_specs=[
            pl.BlockSpec((RB, 128), lambda i, j: (i, 0)),
            pl.BlockSpec((RB, 128), lambda i, j: (NRB + i, 0)),
            pl.BlockSpec((256, 128), lambda i, j: (0, j)),
            pl.BlockSpec((1, 256), lambda i, j: (0, 0)),
            *_deg_specs(),
        ],
        out_specs=pl.BlockSpec((RB, 128), lambda i, j: (j * NRB + i, 0)),
        out_shape=jax.ShapeDtypeStruct((2 * NP, 128), jnp.float32),
    )(z1, z1, wcat, b1r, degs, degs)


def _tc_final(z2, bmur, blsr, degs):
    spec = pl.BlockSpec((RB, 128), lambda i: (i, 0))
    return pl.pallas_call(
        _fin_body,
        grid=(NRB,),
        in_specs=[
            pl.BlockSpec((RB, 128), lambda i: (i, 0)),
            pl.BlockSpec((RB, 128), lambda i: (NRB + i, 0)),
            pl.BlockSpec((1, 128), lambda i: (0, 0)),
            pl.BlockSpec((1, 128), lambda i: (0, 0)),
            pl.BlockSpec((RB, 16), lambda i: (i, 0)),
            pl.BlockSpec((RB, 16), lambda i: (NRB + i, 0)),
        ],
        out_specs=[spec, spec],
        out_shape=[
            jax.ShapeDtypeStruct((NP, 128), jnp.float32),
            jax.ShapeDtypeStruct((NP, 128), jnp.float32),
        ],
    )(z2, z2, bmur, blsr, degs, degs)


def kernel(x, edge_index, W1, b1, Wmu, bmu, Wls, bls):
    pad_e = NCHP * CHUNK - EE
    src2 = jnp.concatenate(
        [edge_index[0], jnp.zeros((pad_e,), jnp.int32)]).reshape(NCHP, CHUNK)
    src3 = jnp.stack([src2, src2 + NP])
    dst2 = jnp.concatenate(
        [edge_index[1], jnp.full((pad_e,), NN, jnp.int32)]).reshape(NCHP, CHUNK)
    xp = jnp.pad(x, ((0, NP - NN), (0, 0)))
    wcat = jnp.concatenate([Wmu, Wls], axis=1)
    b1r = b1.reshape(1, 256)
    bmur = bmu.reshape(1, 128)
    blsr = bls.reshape(1, 128)
    degs = _sc_degree(dst2)
    y1 = _tc_mm1(xp, W1, degs)
    z1 = _sc_propagate(src3, dst2, y1)
    y2 = _tc_mm2(z1, wcat, b1r, degs)
    z2 = _sc_propagate(src3, dst2, y2)
    mu, ls = _tc_final(z2, bmur, blsr, degs)
    return mu[:NN], ls[:NN]
